# baseline (speedup 1.0000x reference)
import jax
import jax.numpy as jnp
from jax.experimental import pallas as pl
from jax.experimental.pallas import tpu as pltpu

N0 = 8192
N1 = 8192
D_EDGE = 128
D_OUT = 128

_BM = 256
_NBLK = N0 // _BM
_NBUF = 4


def _fused_kernel(pw_ref, d_hbm_ref, x1_ref, wt_ref, b_ref, o_ref,
                  d_buf, h_ref, sems):
    i = pl.program_id(0)

    def _start(blk):
        slot = jax.lax.rem(blk, _NBUF)
        pltpu.make_async_copy(
            d_hbm_ref.at[pl.ds(blk * _BM, _BM), :],
            d_buf.at[slot],
            sems.at[slot],
        ).start()

    @pl.when(i == 0)
    def _():
        for j in range(_NBUF):
            _start(j)
        h_ref[...] = jnp.dot(
            x1_ref[...], wt_ref[...],
            precision=jax.lax.Precision.DEFAULT,
            preferred_element_type=jnp.float32,
        ) + b_ref[...]

    slot = jax.lax.rem(i, _NBUF)
    pltpu.make_async_copy(
        d_hbm_ref.at[pl.ds(i * _BM, _BM), :],
        d_buf.at[slot],
        sems.at[slot],
    ).wait()
    acc = jnp.dot(
        d_buf[slot], h_ref[...],
        precision=jax.lax.Precision.DEFAULT,
        preferred_element_type=jnp.float32,
    )
    w = pw_ref[0]
    o_ref[...] = jnp.where(acc >= 0, acc, w * acc)

    @pl.when(i + _NBUF < _NBLK)
    def _():
        _start(i + _NBUF)


def kernel(X0, X1, X2, L0, L1, L2, B2D3, D2B1TD1inv, D1invB1, B2TD2inv, W_e2n, b_e2n, prelu_w):
    grid = (_NBLK,)
    y = pl.pallas_call(
        _fused_kernel,
        grid=grid,
        in_specs=[
            pl.BlockSpec(memory_space=pltpu.SMEM),
            pl.BlockSpec(memory_space=pl.ANY),
            pl.BlockSpec((N1, D_EDGE), lambda i: (0, 0)),
            pl.BlockSpec((D_EDGE, D_OUT), lambda i: (0, 0)),
            pl.BlockSpec((1, D_OUT), lambda i: (0, 0)),
        ],
        out_specs=pl.BlockSpec((_BM, D_OUT), lambda i: (i, 0)),
        out_shape=jax.ShapeDtypeStruct((N0, D_OUT), jnp.float32),
        scratch_shapes=[
            pltpu.VMEM((_NBUF, _BM, N1), jnp.float32),
            pltpu.VMEM((N1, D_OUT), jnp.float32),
            pltpu.SemaphoreType.DMA((_NBUF,)),
        ],
        compiler_params=pltpu.CompilerParams(
            dimension_semantics=("arbitrary",),
        ),
    )(prelu_w, D1invB1, X1, W_e2n.T, b_e2n.reshape(1, D_OUT))
    return y


# speedup vs baseline: 1.0266x; 1.0266x over previous
import jax
import jax.numpy as jnp
from jax.experimental import pallas as pl
from jax.experimental.pallas import tpu as pltpu

N0 = 8192
N1 = 8192
D_EDGE = 128
D_OUT = 128

_BM = 256


def _fused_kernel(pw_ref, d_ref, x1_ref, wt_ref, b_ref, o_ref, h_ref):
    i = pl.program_id(0)

    @pl.when(i == 0)
    def _():
        h_ref[...] = jnp.dot(
            x1_ref[...], wt_ref[...],
            precision=jax.lax.Precision.DEFAULT,
            preferred_element_type=jnp.float32,
        ) + b_ref[...]

    acc = jnp.dot(
        d_ref[...], h_ref[...],
        precision=jax.lax.Precision.DEFAULT,
        preferred_element_type=jnp.float32,
    )
    w = pw_ref[0]
    o_ref[...] = jnp.where(acc >= 0, acc, w * acc)


def kernel(X0, X1, X2, L0, L1, L2, B2D3, D2B1TD1inv, D1invB1, B2TD2inv, W_e2n, b_e2n, prelu_w):
    grid = (N0 // _BM,)
    y = pl.pallas_call(
        _fused_kernel,
        grid=grid,
        in_specs=[
            pl.BlockSpec(memory_space=pltpu.SMEM),
            pl.BlockSpec((_BM, N1), lambda i: (i, 0)),
            pl.BlockSpec((N1, D_EDGE), lambda i: (0, 0)),
            pl.BlockSpec((D_EDGE, D_OUT), lambda i: (0, 0)),
            pl.BlockSpec((1, D_OUT), lambda i: (0, 0)),
        ],
        out_specs=pl.BlockSpec((_BM, D_OUT), lambda i: (i, 0)),
        out_shape=jax.ShapeDtypeStruct((N0, D_OUT), jnp.float32),
        scratch_shapes=[pltpu.VMEM((N1, D_OUT), jnp.float32)],
        compiler_params=pltpu.CompilerParams(
            dimension_semantics=("arbitrary",),
        ),
    )(prelu_w, D1invB1, X1, W_e2n.T, b_e2n.reshape(1, D_OUT))
    return y
